# baseline (speedup 1.0000x reference)
import jax
import jax.numpy as jnp
from jax.experimental import pallas as pl
from jax.experimental.pallas import tpu as pltpu

_THRESHOLD = 0.8
_ROWS = 1000
_LANE_PAD = 128


def _prior_filter_kernel(bi_ref, bjt_ref, boxes_out_ref, keep_out_ref):
    bi = bi_ref[...]
    x1i, y1i, x2i, y2i = (bi[:, 0:1], bi[:, 1:2], bi[:, 2:3], bi[:, 3:4])
    rows = bi.shape[0]
    npl = bjt_ref.shape[1]
    chunk = 1280

    def body(c, acc):
        sl = pl.ds(c * chunk, chunk)
        x1j = bjt_ref[0:1, sl]
        y1j = bjt_ref[1:2, sl]
        x2j = bjt_ref[2:3, sl]
        y2j = bjt_ref[3:4, sl]
        area_j = (x2j - x1j) * (y2j - y1j)
        contained = (
            (x1j >= x1i) & (y1j >= y1i) & (x2j <= x2i) & (y2j <= y2i)
        )
        return acc + jnp.sum(
            jnp.where(contained, jnp.broadcast_to(area_j, contained.shape), 0.0),
            axis=1,
            keepdims=True,
        )

    s = jax.lax.fori_loop(
        0, npl // chunk, body, jnp.zeros((rows, 1), jnp.float32)
    )
    area_i = (x2i - x1i) * (y2i - y1i)
    s = s - area_i
    keep = s <= _THRESHOLD * (area_i + 1e-9)
    keep_out_ref[...] = keep
    boxes_out_ref[...] = bi * keep.astype(bi.dtype)


def kernel(boxes):
    n = boxes.shape[0]
    npad = ((n + _LANE_PAD - 1) // _LANE_PAD) * _LANE_PAD
    bt = jnp.zeros((4, npad), boxes.dtype).at[:, :n].set(boxes.T)
    grid = n // _ROWS
    boxes_out, keep = pl.pallas_call(
        _prior_filter_kernel,
        grid=(grid,),
        in_specs=[
            pl.BlockSpec((_ROWS, 4), lambda i: (i, 0)),
            pl.BlockSpec((4, npad), lambda i: (0, 0)),
        ],
        out_specs=[
            pl.BlockSpec((_ROWS, 4), lambda i: (i, 0)),
            pl.BlockSpec((_ROWS, 1), lambda i: (i, 0)),
        ],
        out_shape=[
            jax.ShapeDtypeStruct((n, 4), boxes.dtype),
            jax.ShapeDtypeStruct((n, 1), jnp.bool_),
        ],
        compiler_params=pltpu.CompilerParams(
            dimension_semantics=("parallel",)
        ),
    )(boxes, bt)
    return boxes_out, keep[:, 0]


# speedup vs baseline: 1.1900x; 1.1900x over previous
import jax
import jax.numpy as jnp
from jax.experimental import pallas as pl
from jax.experimental.pallas import tpu as pltpu

_THRESHOLD = 0.8
_ROWS = 1000
_LANE_PAD = 128


def _prior_filter_kernel(bi_ref, bjt_ref, boxes_out_ref, keep_out_ref):
    bi = bi_ref[...]
    x1i, y1i, x2i, y2i = (bi[:, 0:1], bi[:, 1:2], bi[:, 2:3], bi[:, 3:4])
    x1j = bjt_ref[0:1, :]
    y1j = bjt_ref[1:2, :]
    x2j = bjt_ref[2:3, :]
    y2j = bjt_ref[3:4, :]
    area_j = (x2j - x1j) * (y2j - y1j)
    contained = (
        (x1j >= x1i) & (y1j >= y1i) & (x2j <= x2i) & (y2j <= y2i)
    )
    s = jnp.sum(
        jnp.where(contained, jnp.broadcast_to(area_j, contained.shape), 0.0),
        axis=1,
        keepdims=True,
    )
    area_i = (x2i - x1i) * (y2i - y1i)
    s = s - area_i
    keep = jnp.where(
        s <= _THRESHOLD * (area_i + 1e-9), 1.0, 0.0
    )
    keep_out_ref[...] = keep
    boxes_out_ref[...] = bi * keep


def kernel(boxes):
    n = boxes.shape[0]
    npad = ((n + _LANE_PAD - 1) // _LANE_PAD) * _LANE_PAD
    bt = jnp.zeros((4, npad), boxes.dtype).at[:, :n].set(boxes.T)
    grid = n // _ROWS
    boxes_out, keep = pl.pallas_call(
        _prior_filter_kernel,
        grid=(grid,),
        in_specs=[
            pl.BlockSpec((_ROWS, 4), lambda i: (i, 0)),
            pl.BlockSpec((4, npad), lambda i: (0, 0)),
        ],
        out_specs=[
            pl.BlockSpec((_ROWS, 4), lambda i: (i, 0)),
            pl.BlockSpec((_ROWS, 1), lambda i: (i, 0)),
        ],
        out_shape=[
            jax.ShapeDtypeStruct((n, 4), boxes.dtype),
            jax.ShapeDtypeStruct((n, 1), jnp.float32),
        ],
        compiler_params=pltpu.CompilerParams(
            dimension_semantics=("parallel",)
        ),
    )(boxes, bt)
    return boxes_out, keep[:, 0] > 0.5
